# baseline (speedup 1.0000x reference)
import functools

import jax
import jax.numpy as jnp
from jax import lax
from jax.experimental import pallas as pl
from jax.experimental.pallas import tpu as pltpu
from jax.experimental.pallas import tpu_sc as plsc

VOCAB = 64
EMBED = 16
B = 4
T = 8192
N = B * T
PAIRS = B * (T - 1)
IDPAD = 8

NC = 2
NS = 16
NW = NC * NS
TOK_W = N // NW
CHUNK = 128
LANES = 16

TBLK = 2048


def _tables_body(we_ref, wp_ref, wpt_ref, wet_ref, b_ref, bc_ref,
                 lt_ref, nll_ref):
    l_tab = (
        jnp.dot(we_ref[...], wp_ref[...], preferred_element_type=jnp.float32)
        + b_ref[...]
    )
    lt_ref[...] = (
        jnp.dot(wpt_ref[...], wet_ref[...], preferred_element_type=jnp.float32)
        + bc_ref[...]
    )
    m = jnp.max(l_tab, axis=1, keepdims=True)
    lse = jnp.log(jnp.sum(jnp.exp(l_tab - m), axis=1, keepdims=True)) + m
    nll_ref[...] = jnp.concatenate(
        [lse - l_tab, jnp.zeros((VOCAB, VOCAB), jnp.float32)], axis=1
    )


_tables = pl.pallas_call(
    _tables_body,
    out_shape=[
        jax.ShapeDtypeStruct((VOCAB, VOCAB), jnp.float32),
        jax.ShapeDtypeStruct((VOCAB, 2 * VOCAB), jnp.float32),
    ],
)


def _logits_body(ids_ref, lt_ref, out_ref):
    ids_blk = ids_ref[0, 0]
    onehot_t = jnp.where(
        lax.broadcasted_iota(jnp.int32, (VOCAB, TBLK), 0) == ids_blk[None, :],
        1.0,
        0.0,
    )
    out_ref[...] = lax.dot_general(
        lt_ref[...],
        onehot_t,
        (((1,), (0,)), ((), ())),
        preferred_element_type=jnp.float32,
        precision=lax.Precision.HIGHEST,
    )[None]


_logits = pl.pallas_call(
    _logits_body,
    grid=(B, T // TBLK),
    in_specs=[
        pl.BlockSpec((1, 1, TBLK), lambda b, t: (b, 0, t)),
        pl.BlockSpec((VOCAB, VOCAB), lambda b, t: (0, 0)),
    ],
    out_specs=pl.BlockSpec((1, VOCAB, TBLK), lambda b, t: (b, 0, t)),
    out_shape=jax.ShapeDtypeStruct((B, VOCAB, T), jnp.float32),
)


_mesh = plsc.VectorSubcoreMesh(core_axis_name="c", subcore_axis_name="s")


@functools.partial(
    pl.kernel,
    mesh=_mesh,
    out_type=jax.ShapeDtypeStruct((NC, LANES), jnp.float32),
    scratch_types=[
        pltpu.VMEM((TOK_W + IDPAD,), jnp.int32),
        pltpu.VMEM((TOK_W,), jnp.int32),
        pltpu.VMEM((TOK_W,), jnp.float32),
        pltpu.VMEM((LANES,), jnp.float32),
        pltpu.VMEM((NS * LANES,), jnp.float32),
        pltpu.VMEM_SHARED((NS * LANES,), jnp.float32),
        pltpu.SemaphoreType.DMA,
    ],
)
def _sc_loss(ids_hbm, nll_hbm, loss_hbm,
             ids_v, p_v, vals_v, stage_v, part_v, part_sh, lsem):
    cid = lax.axis_index("c")
    sid = lax.axis_index("s")
    wid = sid * NC + cid
    tbase = wid * TOK_W

    pltpu.sync_copy(ids_hbm.at[pl.ds(tbase, TOK_W + IDPAD)], ids_v)

    def ibody(i, carry):
        c = ids_v[pl.ds(i * LANES, LANES)]
        n = ids_v[pl.ds(i * LANES + 1, LANES)]
        p_v[pl.ds(i * LANES, LANES)] = c * (2 * VOCAB) + n
        return carry

    lax.fori_loop(0, TOK_W // LANES, ibody, 0, unroll=8)

    lhandles = []
    for j in range(TOK_W // CHUNK):
        lhandles.append(
            pltpu.async_copy(
                nll_hbm.at[p_v.at[pl.ds(j * CHUNK, CHUNK)]],
                vals_v.at[pl.ds(j * CHUNK, CHUNK)],
                lsem,
            )
        )
    for h in lhandles:
        h.wait()

    def abody(i, acc):
        t = tbase + i * LANES + lax.iota(jnp.int32, LANES)
        vals = vals_v[pl.ds(i * LANES, LANES)]
        return acc + jnp.where((t & (T - 1)) != (T - 1), vals, 0.0)

    acc = lax.fori_loop(
        0, TOK_W // LANES, abody, jnp.zeros((LANES,), jnp.float32)
    )
    stage_v[...] = acc
    pltpu.sync_copy(stage_v, part_sh.at[pl.ds(sid * LANES, LANES)])

    plsc.subcore_barrier()

    @pl.when(sid == 0)
    def _loss_core_sum():
        pltpu.sync_copy(part_sh, part_v)

        def body(i, acc):
            return acc + part_v[pl.ds(i * LANES, LANES)]

        tot = lax.fori_loop(0, NS, body, jnp.zeros((LANES,), jnp.float32))
        total = tot[0]
        for i in range(1, LANES):
            total = total + tot[i]
        stage_v[...] = jnp.zeros((LANES,), jnp.float32) + total * (1.0 / PAIRS)
        pltpu.sync_copy(stage_v, loss_hbm.at[cid])


def kernel(input_ids, W_embed, W_proj, b_proj):
    ids = input_ids.astype(jnp.int32)
    lt_tab, nll_tab = _tables(
        W_embed, W_proj, W_proj.T, W_embed.T,
        b_proj.reshape(1, VOCAB), b_proj.reshape(VOCAB, 1),
    )

    ids_flat = jnp.concatenate([ids.reshape(-1), jnp.zeros((IDPAD,), jnp.int32)])
    loss2 = _sc_loss(ids_flat, nll_tab.reshape(-1))
    logits_vt = _logits(ids.reshape(B, 1, T), lt_tab)
    return loss2[0, 0] + loss2[1, 0], jnp.swapaxes(logits_vt, 1, 2)


# speedup vs baseline: 1.0759x; 1.0759x over previous
import functools

import jax
import jax.numpy as jnp
from jax import lax
from jax.experimental import pallas as pl
from jax.experimental.pallas import tpu as pltpu
from jax.experimental.pallas import tpu_sc as plsc

VOCAB = 64
EMBED = 16
B = 4
T = 8192
N = B * T
PAIRS = B * (T - 1)
IDPAD = 8

NC = 2
NS = 16
NW = NC * NS
TOK_W = N // NW
CHUNK = 128
LANES = 16

TBLK = 2048


def _tables_body(we_ref, wp_ref, b_ref, lt_ref, nll_ref):
    l_tab = (
        jnp.dot(we_ref[...], wp_ref[...], preferred_element_type=jnp.float32)
        + b_ref[...]
    )
    lt_ref[...] = l_tab.T
    m = jnp.max(l_tab, axis=1, keepdims=True)
    lse = jnp.log(jnp.sum(jnp.exp(l_tab - m), axis=1, keepdims=True)) + m
    nll_ref[...] = jnp.concatenate(
        [lse - l_tab, jnp.zeros((VOCAB, VOCAB), jnp.float32)], axis=1
    )


_tables = pl.pallas_call(
    _tables_body,
    out_shape=[
        jax.ShapeDtypeStruct((VOCAB, VOCAB), jnp.float32),
        jax.ShapeDtypeStruct((VOCAB, 2 * VOCAB), jnp.float32),
    ],
)


def _logits_body(ids_ref, lt_ref, out_ref):
    ids_blk = ids_ref[0, 0]
    onehot_t = jnp.where(
        lax.broadcasted_iota(jnp.int32, (VOCAB, TBLK), 0) == ids_blk[None, :],
        1.0,
        0.0,
    )
    out_ref[...] = lax.dot_general(
        lt_ref[...],
        onehot_t,
        (((1,), (0,)), ((), ())),
        preferred_element_type=jnp.float32,
        precision=lax.Precision.HIGHEST,
    )[None]


_logits = pl.pallas_call(
    _logits_body,
    grid=(B, T // TBLK),
    in_specs=[
        pl.BlockSpec((1, 1, TBLK), lambda b, t: (b, 0, t)),
        pl.BlockSpec((VOCAB, VOCAB), lambda b, t: (0, 0)),
    ],
    out_specs=pl.BlockSpec((1, VOCAB, TBLK), lambda b, t: (b, 0, t)),
    out_shape=jax.ShapeDtypeStruct((B, VOCAB, T), jnp.float32),
)


_mesh = plsc.VectorSubcoreMesh(core_axis_name="c", subcore_axis_name="s")


@functools.partial(
    pl.kernel,
    mesh=_mesh,
    out_type=jax.ShapeDtypeStruct((NC, LANES), jnp.float32),
    scratch_types=[
        pltpu.VMEM((TOK_W + IDPAD,), jnp.int32),
        pltpu.VMEM((TOK_W,), jnp.int32),
        pltpu.VMEM((TOK_W,), jnp.float32),
        pltpu.VMEM((LANES,), jnp.float32),
        pltpu.VMEM((NS * LANES,), jnp.float32),
        pltpu.VMEM_SHARED((NS * LANES,), jnp.float32),
        pltpu.SemaphoreType.DMA,
    ],
)
def _sc_loss(ids_hbm, nll_hbm, loss_hbm,
             ids_v, p_v, vals_v, stage_v, part_v, part_sh, lsem):
    cid = lax.axis_index("c")
    sid = lax.axis_index("s")
    wid = sid * NC + cid
    tbase = wid * TOK_W

    pltpu.sync_copy(ids_hbm.at[pl.ds(tbase, TOK_W + IDPAD)], ids_v)

    for i in range(TOK_W // LANES):
        c = ids_v[pl.ds(i * LANES, LANES)]
        n = ids_v[pl.ds(i * LANES + 1, LANES)]
        p_v[pl.ds(i * LANES, LANES)] = c * (2 * VOCAB) + n

    lhandles = []
    for j in range(TOK_W // CHUNK):
        lhandles.append(
            pltpu.async_copy(
                nll_hbm.at[p_v.at[pl.ds(j * CHUNK, CHUNK)]],
                vals_v.at[pl.ds(j * CHUNK, CHUNK)],
                lsem,
            )
        )
    for h in lhandles:
        h.wait()

    def abody(i, acc):
        t = tbase + i * LANES + lax.iota(jnp.int32, LANES)
        vals = vals_v[pl.ds(i * LANES, LANES)]
        return acc + jnp.where((t & (T - 1)) != (T - 1), vals, 0.0)

    acc = lax.fori_loop(
        0, TOK_W // LANES, abody, jnp.zeros((LANES,), jnp.float32)
    )
    stage_v[...] = acc
    pltpu.sync_copy(stage_v, part_sh.at[pl.ds(sid * LANES, LANES)])

    plsc.subcore_barrier()

    @pl.when(sid == 0)
    def _loss_core_sum():
        pltpu.sync_copy(part_sh, part_v)

        def body(i, acc):
            return acc + part_v[pl.ds(i * LANES, LANES)]

        tot = lax.fori_loop(0, NS, body, jnp.zeros((LANES,), jnp.float32))
        total = tot[0]
        for i in range(1, LANES):
            total = total + tot[i]
        stage_v[...] = jnp.zeros((LANES,), jnp.float32) + total * (1.0 / PAIRS)
        pltpu.sync_copy(stage_v, loss_hbm.at[cid])


def kernel(input_ids, W_embed, W_proj, b_proj):
    ids = input_ids.astype(jnp.int32)
    lt_tab, nll_tab = _tables(W_embed, W_proj, b_proj.reshape(1, VOCAB))

    ids_flat = jnp.concatenate([ids.reshape(-1), jnp.zeros((IDPAD,), jnp.int32)])
    loss2 = _sc_loss(ids_flat, nll_tab.reshape(-1))
    logits_vt = _logits(ids.reshape(B, 1, T), lt_tab)
    return loss2[0, 0] + loss2[1, 0], jnp.swapaxes(logits_vt, 1, 2)


# speedup vs baseline: 1.2655x; 1.1763x over previous
import functools

import jax
import jax.numpy as jnp
from jax import lax
from jax.experimental import pallas as pl
from jax.experimental.pallas import tpu as pltpu
from jax.experimental.pallas import tpu_sc as plsc

VOCAB = 64
EMBED = 16
B = 4
T = 8192
N = B * T
PAIRS = B * (T - 1)
IDPAD = 8

NC = 2
NS = 16
NW = NC * NS
TOK_W = N // NW
CHUNK = 128
LANES = 16

TBLK = 2048


def _tables_body(we_ref, wp_ref, b_ref, lt_ref, nll_ref):
    l_tab = (
        jnp.dot(we_ref[...], wp_ref[...], preferred_element_type=jnp.float32)
        + b_ref[...]
    )
    lt_ref[...] = l_tab.T
    m = jnp.max(l_tab, axis=1, keepdims=True)
    lse = jnp.log(jnp.sum(jnp.exp(l_tab - m), axis=1, keepdims=True)) + m
    nll_ref[...] = jnp.concatenate(
        [lse - l_tab, jnp.zeros((VOCAB, VOCAB), jnp.float32)], axis=1
    )


_tables = pl.pallas_call(
    _tables_body,
    out_shape=[
        jax.ShapeDtypeStruct((VOCAB, VOCAB), jnp.float32),
        jax.ShapeDtypeStruct((VOCAB, 2 * VOCAB), jnp.float32),
    ],
)


def _logits_body(ids_ref, lt_ref, out_ref):
    for b in range(B):
        onehot_t = jnp.where(
            lax.broadcasted_iota(jnp.int32, (VOCAB, TBLK), 0)
            == ids_ref[b][None, :],
            1.0,
            0.0,
        )
        out_ref[b] = lax.dot_general(
            lt_ref[...],
            onehot_t,
            (((1,), (0,)), ((), ())),
            preferred_element_type=jnp.float32,
            precision=lax.Precision.HIGHEST,
        )


_logits = pl.pallas_call(
    _logits_body,
    grid=(T // TBLK,),
    in_specs=[
        pl.BlockSpec((B, TBLK), lambda t: (0, t)),
        pl.BlockSpec((VOCAB, VOCAB), lambda t: (0, 0)),
    ],
    out_specs=pl.BlockSpec((B, VOCAB, TBLK), lambda t: (0, 0, t)),
    out_shape=jax.ShapeDtypeStruct((B, VOCAB, T), jnp.float32),
)


_mesh = plsc.VectorSubcoreMesh(core_axis_name="c", subcore_axis_name="s")


@functools.partial(
    pl.kernel,
    mesh=_mesh,
    out_type=jax.ShapeDtypeStruct((NC, LANES), jnp.float32),
    scratch_types=[
        pltpu.VMEM((TOK_W + IDPAD,), jnp.int32),
        pltpu.VMEM((TOK_W,), jnp.int32),
        pltpu.VMEM((TOK_W,), jnp.float32),
        pltpu.VMEM((LANES,), jnp.float32),
        pltpu.VMEM((NS * LANES,), jnp.float32),
        pltpu.VMEM_SHARED((NS * LANES,), jnp.float32),
        pltpu.SemaphoreType.DMA,
    ],
)
def _sc_loss(ids_hbm, nll_hbm, loss_hbm,
             ids_v, p_v, vals_v, stage_v, part_v, part_sh, lsem):
    cid = lax.axis_index("c")
    sid = lax.axis_index("s")
    wid = sid * NC + cid
    tbase = wid * TOK_W

    pltpu.sync_copy(ids_hbm.at[pl.ds(tbase, TOK_W + IDPAD)], ids_v)

    def ibody(i, carry):
        c = ids_v[pl.ds(i * LANES, LANES)]
        n = ids_v[pl.ds(i * LANES + 1, LANES)]
        p_v[pl.ds(i * LANES, LANES)] = c * (2 * VOCAB) + n
        return carry

    lax.fori_loop(0, TOK_W // LANES, ibody, 0)

    lhandles = []
    for j in range(TOK_W // CHUNK):
        lhandles.append(
            pltpu.async_copy(
                nll_hbm.at[p_v.at[pl.ds(j * CHUNK, CHUNK)]],
                vals_v.at[pl.ds(j * CHUNK, CHUNK)],
                lsem,
            )
        )
    for h in lhandles:
        h.wait()

    def abody(i, acc):
        t = tbase + i * LANES + lax.iota(jnp.int32, LANES)
        vals = vals_v[pl.ds(i * LANES, LANES)]
        return acc + jnp.where((t & (T - 1)) != (T - 1), vals, 0.0)

    acc = lax.fori_loop(
        0, TOK_W // LANES, abody, jnp.zeros((LANES,), jnp.float32)
    )
    stage_v[...] = acc
    pltpu.sync_copy(stage_v, part_sh.at[pl.ds(sid * LANES, LANES)])

    plsc.subcore_barrier()

    @pl.when(sid == 0)
    def _loss_core_sum():
        pltpu.sync_copy(part_sh, part_v)

        def body(i, acc):
            return acc + part_v[pl.ds(i * LANES, LANES)]

        tot = lax.fori_loop(0, NS, body, jnp.zeros((LANES,), jnp.float32))
        total = tot[0]
        for i in range(1, LANES):
            total = total + tot[i]
        stage_v[...] = jnp.zeros((LANES,), jnp.float32) + total * (1.0 / PAIRS)
        pltpu.sync_copy(stage_v, loss_hbm.at[cid])


def kernel(input_ids, W_embed, W_proj, b_proj):
    ids = input_ids.astype(jnp.int32)
    lt_tab, nll_tab = _tables(W_embed, W_proj, b_proj.reshape(1, VOCAB))

    ids_flat = jnp.concatenate([ids.reshape(-1), jnp.zeros((IDPAD,), jnp.int32)])
    loss2 = _sc_loss(ids_flat, nll_tab.reshape(-1))
    logits_vt = _logits(ids, lt_tab)
    return loss2[0, 0] + loss2[1, 0], jnp.swapaxes(logits_vt, 1, 2)
